# baseline (speedup 1.0000x reference)
import jax
import jax.numpy as jnp
from jax import lax
from jax.experimental import pallas as pl
from jax.experimental.pallas import tpu as pltpu, tpu_sc as plsc

N_NODES = 10000
N_PAD = 10240
E = 160000
F = 256
FH = 128
NC, NS = 2, 16
EPW = E // (NC * NS)
EPS = E // NS
CH = 80
NCHUNK = EPS // CH
RPT = N_PAD // NS
RCH = 16
VEC = 16


def _deg_body(dst_hbm, degp_hbm, dbuf, cnt, sem):
    c = lax.axis_index("c")
    s = lax.axis_index("s")
    wid = s * NC + c
    ones = jnp.ones((VEC,), jnp.float32)
    lane = lax.iota(jnp.int32, VEC)

    def zero(i, _):
        cnt[pl.ds(pl.multiple_of(i * VEC, VEC), VEC)] = jnp.zeros(
            (VEC,), jnp.float32)
        return 0
    lax.fori_loop(0, N_PAD // VEC, zero, 0)

    pltpu.sync_copy(dst_hbm.at[pl.ds(wid * EPW, EPW)], dbuf)

    nfull = EPW // VEC
    def body(i, _):
        idx = dbuf[pl.ds(pl.multiple_of(i * VEC, VEC), VEC)]
        plsc.addupdate_scatter(cnt, [idx], ones)
        return 0
    lax.fori_loop(0, nfull, body, 0)

    tail = EPW - nfull * VEC
    idx = dbuf[pl.ds(nfull * VEC - (VEC - tail), VEC)]
    mask = lane >= (VEC - tail)
    plsc.addupdate_scatter(cnt, [idx], ones, mask=mask)

    pltpu.sync_copy(cnt, degp_hbm.at[wid])


def _deg_partial(dst):
    mesh = plsc.VectorSubcoreMesh(core_axis_name="c", subcore_axis_name="s")
    return pl.kernel(
        _deg_body,
        out_type=jax.ShapeDtypeStruct((NC * NS, N_PAD), jnp.float32),
        mesh=mesh,
        scratch_types=[
            pltpu.VMEM((EPW,), jnp.int32),
            pltpu.VMEM((N_PAD,), jnp.float32),
            pltpu.SemaphoreType.DMA,
        ],
        compiler_params=pltpu.CompilerParams(needs_layout_passes=False),
        name="gcn_degree_sc",
    )(dst)


RB = 400

def _dis_body(degp_ref, dis_ref, disx_ref):
    deg = jnp.sum(degp_ref[...], axis=0)
    dis = jnp.where(deg > 0.0, lax.rsqrt(jnp.maximum(deg, 1e-12)), 0.0)
    dis_ref[...] = dis[:, None]
    disx_ref[...] = jnp.broadcast_to(dis[:, None], (N_PAD, VEC))


def _dis_from_partials(degp):
    return pl.pallas_call(
        _dis_body,
        out_shape=[
            jax.ShapeDtypeStruct((N_PAD, 1), jnp.float32),
            jax.ShapeDtypeStruct((N_PAD, VEC), jnp.float32),
        ],
        name="gcn_dis_tc",
    )(degp)


def _mm_body(x_ref, w_ref, dis_ref, hp_ref):
    h = jnp.dot(x_ref[...], w_ref[...], preferred_element_type=jnp.float32)
    hp_ref[0] = h * dis_ref[...]


def _matmul_scale(x, W, dis):
    grid = (N_NODES // RB, NC)
    return pl.pallas_call(
        _mm_body,
        grid=grid,
        in_specs=[
            pl.BlockSpec((RB, F), lambda i, c: (i, 0)),
            pl.BlockSpec((F, FH), lambda i, c: (0, c)),
            pl.BlockSpec((RB, 1), lambda i, c: (i, 0)),
        ],
        out_specs=pl.BlockSpec((1, RB, FH), lambda i, c: (c, i, 0)),
        out_shape=jax.ShapeDtypeStruct((NC, N_NODES, FH), jnp.float32),
        name="gcn_matmul_scale_tc",
    )(x, W, dis)


NBUF = 2


def _scale_rows(stage, disxc, b_vmem, nrows):
    for i in range(nrows):
        dv = disxc[i]
        for v in range(FH // VEC):
            sl = pl.ds(v * VEC, VEC)
            stage[i, sl] = stage[i, sl] * dv + b_vmem[sl]


def _scatter_body(hp_hbm, src_hbm, dst_hbm, disx_hbm, b_hbm, out_hbm,
                  src_all, dst_all, idx_b, dst_b, rows_b, stage, disxc,
                  b_vmem, acc_sh, sems):
    c = lax.axis_index("c")
    s = lax.axis_index("s")
    base = s * EPS
    off = (c * N_NODES).astype(jnp.int32)

    pltpu.sync_copy(src_hbm.at[pl.ds(base, EPS)], src_all)
    pltpu.sync_copy(dst_hbm.at[pl.ds(base, EPS)], dst_all)

    def zstage(i, _):
        for v in range(FH // VEC):
            stage[i, pl.ds(v * VEC, VEC)] = jnp.zeros((VEC,), jnp.float32)
        return 0
    lax.fori_loop(0, RCH, zstage, 0)
    row0 = s * RPT
    for k in range(RPT // RCH):
        pltpu.sync_copy(stage, acc_sh.at[pl.ds(row0 + k * RCH, RCH)])
    plsc.subcore_barrier()

    def prep_issue(jj, b):
        e0 = pl.multiple_of(jj * CH, VEC)
        for v in range(CH // VEC):
            idx_b[b][pl.ds(v * VEC, VEC)] = (
                src_all[pl.ds(e0 + v * VEC, VEC)] + off)
            dst_b[b][pl.ds(v * VEC, VEC)] = dst_all[pl.ds(e0 + v * VEC, VEC)]
        pltpu.async_copy(hp_hbm.at[idx_b[b]], rows_b[b], sems[b])

    for b in range(NBUF):
        prep_issue(jnp.int32(b), b)

    def step(t, _):
        for b in range(NBUF):
            jj = t * NBUF + b
            pltpu.make_async_copy(
                hp_hbm.at[idx_b[b]], rows_b[b], sems[b]).wait()
            pltpu.sync_copy(rows_b[b], acc_sh.at[dst_b[b]], add=True)
            nxt = jj + NBUF

            @pl.when(nxt < NCHUNK)
            def _():
                prep_issue(nxt, b)
        return 0
    lax.fori_loop(0, NCHUNK // NBUF, step, 0)
    last = NCHUNK - NCHUNK % NBUF
    for b in range(NCHUNK % NBUF):
        pltpu.make_async_copy(hp_hbm.at[idx_b[b]], rows_b[b], sems[b]).wait()
        pltpu.sync_copy(rows_b[b], acc_sh.at[dst_b[b]], add=True)
    del last

    plsc.subcore_barrier()

    pltpu.sync_copy(b_hbm.at[pl.ds(c * FH, FH)], b_vmem)
    ncol = pl.multiple_of(c * FH, FH)
    nk = jnp.where(s == NS - 1, (N_NODES - (NS - 1) * RPT) // RCH,
                   RPT // RCH)

    def wchunk(k, _):
        r = pl.multiple_of(row0 + k * RCH, 8)
        pltpu.sync_copy(acc_sh.at[pl.ds(r, RCH)], stage)
        pltpu.sync_copy(disx_hbm.at[pl.ds(r, RCH)], disxc)
        _scale_rows(stage, disxc, b_vmem, RCH)
        pltpu.sync_copy(stage, out_hbm.at[pl.ds(r, RCH), pl.ds(ncol, FH)])
        return 0
    lax.fori_loop(0, nk, wchunk, 0)


def _edge_scatter(hp_flat, src, dst, disx, b):
    mesh = plsc.VectorSubcoreMesh(core_axis_name="c", subcore_axis_name="s")
    return pl.kernel(
        _scatter_body,
        out_type=jax.ShapeDtypeStruct((N_NODES, F), jnp.float32),
        mesh=mesh,
        scratch_types=[
            pltpu.VMEM((EPS,), jnp.int32),
            pltpu.VMEM((EPS,), jnp.int32),
            [pltpu.VMEM((CH,), jnp.int32) for _ in range(NBUF)],
            [pltpu.VMEM((CH,), jnp.int32) for _ in range(NBUF)],
            [pltpu.VMEM((CH, FH), jnp.float32) for _ in range(NBUF)],
            pltpu.VMEM((RCH, FH), jnp.float32),
            pltpu.VMEM((RCH, VEC), jnp.float32),
            pltpu.VMEM((FH,), jnp.float32),
            pltpu.VMEM_SHARED((N_PAD, FH), jnp.float32),
            [pltpu.SemaphoreType.DMA for _ in range(NBUF)],
        ],
        compiler_params=pltpu.CompilerParams(needs_layout_passes=False),
        name="gcn_edge_scatter_sc",
    )(hp_flat, src, dst, disx, b)


def kernel(x, edge_index, W, b):
    src = edge_index[0].astype(jnp.int32)
    dst = edge_index[1].astype(jnp.int32)
    degp = _deg_partial(dst)
    dis, disx = _dis_from_partials(degp)
    hp = _matmul_scale(x, W, dis)
    return _edge_scatter(hp.reshape(NC * N_NODES, FH), src, dst, disx, b)


# speedup vs baseline: 1.1250x; 1.1250x over previous
import jax
import jax.numpy as jnp
from jax import lax
from jax.experimental import pallas as pl
from jax.experimental.pallas import tpu as pltpu, tpu_sc as plsc

N_NODES = 10000
N_PAD = 10240
E = 160000
F = 256
FH = 128
NC, NS = 2, 16
EPW = E // (NC * NS)
EPS = E // NS
CH = 80
NCHUNK = EPS // CH
RPT = N_PAD // NS
RCH = 16
VEC = 16
NB = 3
LOOK = 2


def _deg_body(dst_hbm, degp_hbm, dbuf, cnt, sem):
    c = lax.axis_index("c")
    s = lax.axis_index("s")
    wid = s * NC + c
    ones = jnp.ones((VEC,), jnp.float32)
    lane = lax.iota(jnp.int32, VEC)

    def zero(i, _):
        cnt[pl.ds(pl.multiple_of(i * VEC, VEC), VEC)] = jnp.zeros(
            (VEC,), jnp.float32)
        return 0
    lax.fori_loop(0, N_PAD // VEC, zero, 0)

    pltpu.sync_copy(dst_hbm.at[pl.ds(wid * EPW, EPW)], dbuf)

    nfull = EPW // VEC
    def body(i, _):
        idx = dbuf[pl.ds(pl.multiple_of(i * VEC, VEC), VEC)]
        plsc.addupdate_scatter(cnt, [idx], ones)
        return 0
    lax.fori_loop(0, nfull, body, 0)

    tail = EPW - nfull * VEC
    idx = dbuf[pl.ds(nfull * VEC - (VEC - tail), VEC)]
    mask = lane >= (VEC - tail)
    plsc.addupdate_scatter(cnt, [idx], ones, mask=mask)

    pltpu.sync_copy(cnt, degp_hbm.at[wid])


def _deg_partial(dst):
    mesh = plsc.VectorSubcoreMesh(core_axis_name="c", subcore_axis_name="s")
    return pl.kernel(
        _deg_body,
        out_type=jax.ShapeDtypeStruct((NC * NS, N_PAD), jnp.float32),
        mesh=mesh,
        scratch_types=[
            pltpu.VMEM((EPW,), jnp.int32),
            pltpu.VMEM((N_PAD,), jnp.float32),
            pltpu.SemaphoreType.DMA,
        ],
        compiler_params=pltpu.CompilerParams(needs_layout_passes=False),
        name="gcn_degree_sc",
    )(dst)


def _dis_body(degp_ref, dis_ref):
    deg = jnp.sum(degp_ref[...], axis=0)
    dis = jnp.where(deg > 0.0, lax.rsqrt(jnp.maximum(deg, 1e-12)), 0.0)
    dis_ref[...] = dis[:, None]


def _dis_from_partials(degp):
    return pl.pallas_call(
        _dis_body,
        out_shape=jax.ShapeDtypeStruct((N_PAD, 1), jnp.float32),
        name="gcn_dis_tc",
    )(degp)


RB = 400

def _mm_body(x_ref, w_ref, dis_ref, hp_ref):
    h = jnp.dot(x_ref[...], w_ref[...], preferred_element_type=jnp.float32)
    hp_ref[0] = h * dis_ref[...]


def _matmul_scale(x, W, dis):
    grid = (N_NODES // RB, NC)
    return pl.pallas_call(
        _mm_body,
        grid=grid,
        in_specs=[
            pl.BlockSpec((RB, F), lambda i, c: (i, 0)),
            pl.BlockSpec((F, FH), lambda i, c: (0, c)),
            pl.BlockSpec((RB, 1), lambda i, c: (i, 0)),
        ],
        out_specs=pl.BlockSpec((1, RB, FH), lambda i, c: (c, i, 0)),
        out_shape=jax.ShapeDtypeStruct((NC, N_NODES, FH), jnp.float32),
        name="gcn_matmul_scale_tc",
    )(x, W, dis)


def _scatter_body(hp_hbm, src_hbm, dst_hbm, acc_hbm,
                  src_all, idx_b, dst_b, rows_b, stage, acc_sh,
                  gsem, dsem, ssem):
    c = lax.axis_index("c")
    s = lax.axis_index("s")
    base = s * EPS
    off = (c * N_NODES).astype(jnp.int32)

    pltpu.sync_copy(src_hbm.at[pl.ds(base, EPS)], src_all)

    for v in range(FH // VEC):
        stage[0, pl.ds(v * VEC, VEC)] = jnp.zeros((VEC,), jnp.float32)
    def zstage(i, _):
        for v in range(FH // VEC):
            stage[i, pl.ds(v * VEC, VEC)] = jnp.zeros((VEC,), jnp.float32)
        return 0
    lax.fori_loop(1, RCH, zstage, 0)
    row0 = s * RPT
    for k in range(RPT // RCH):
        pltpu.sync_copy(stage, acc_sh.at[pl.ds(row0 + k * RCH, RCH)])
    plsc.subcore_barrier()

    def prep_issue(jj, b):
        e0 = pl.multiple_of(jj * CH, VEC)
        for v in range(CH // VEC):
            idx_b[b][pl.ds(v * VEC, VEC)] = (
                src_all[pl.ds(e0 + v * VEC, VEC)] + off)
        pltpu.async_copy(dst_hbm.at[pl.ds(base + e0, CH)], dst_b[b], dsem[b])
        pltpu.async_copy(hp_hbm.at[idx_b[b]], rows_b[b], gsem[b])

    for j in range(LOOK):
        prep_issue(jnp.int32(j), j)

    def consume(jj, b):
        pltpu.make_async_copy(hp_hbm.at[idx_b[b]], rows_b[b], gsem[b]).wait()
        pltpu.make_async_copy(
            dst_hbm.at[pl.ds(base, CH)], dst_b[b], dsem[b]).wait()
        pltpu.async_copy(rows_b[b], acc_sh.at[dst_b[b]], ssem[b], add=True)

    def step(t, _):
        for b0 in range(NB):
            jj = t * NB + b0

            @pl.when(jj < NCHUNK)
            def _():
                consume(jj, b0)
                nxt = jj + LOOK

                @pl.when(nxt < NCHUNK)
                def _():
                    b2 = (b0 + LOOK) % NB

                    @pl.when(nxt >= NB)
                    def _():
                        pltpu.make_async_copy(
                            rows_b[b2], acc_sh.at[dst_b[b2]],
                            ssem[b2]).wait()
                    prep_issue(nxt, b2)
        return 0
    lax.fori_loop(0, (NCHUNK + NB - 1) // NB, step, 0)

    for b in range(NB):
        pltpu.make_async_copy(rows_b[b], acc_sh.at[dst_b[b]], ssem[b]).wait()

    plsc.subcore_barrier()
    for k in range(RPT // RCH):
        r = row0 + k * RCH
        pltpu.sync_copy(acc_sh.at[pl.ds(r, RCH)], stage)
        pltpu.sync_copy(stage, acc_hbm.at[c, pl.ds(r, RCH)])


def _edge_scatter(hp_flat, src, dst):
    mesh = plsc.VectorSubcoreMesh(core_axis_name="c", subcore_axis_name="s")
    return pl.kernel(
        _scatter_body,
        out_type=jax.ShapeDtypeStruct((NC, N_PAD, FH), jnp.float32),
        mesh=mesh,
        scratch_types=[
            pltpu.VMEM((EPS,), jnp.int32),
            [pltpu.VMEM((CH,), jnp.int32) for _ in range(NB)],
            [pltpu.VMEM((CH,), jnp.int32) for _ in range(NB)],
            [pltpu.VMEM((CH, FH), jnp.float32) for _ in range(NB)],
            pltpu.VMEM((RCH, FH), jnp.float32),
            pltpu.VMEM_SHARED((N_PAD, FH), jnp.float32),
            [pltpu.SemaphoreType.DMA for _ in range(NB)],
            [pltpu.SemaphoreType.DMA for _ in range(NB)],
            [pltpu.SemaphoreType.DMA for _ in range(NB)],
        ],
        compiler_params=pltpu.CompilerParams(needs_layout_passes=False),
        name="gcn_edge_scatter_sc",
    )(hp_flat, src, dst)


def _out_body(acc_ref, dis_ref, b_ref, out_ref):
    out_ref[...] = acc_ref[0] * dis_ref[...] + b_ref[...][None, :]


def _out_scale(acc, dis, b):
    grid = (N_NODES // RB, NC)
    return pl.pallas_call(
        _out_body,
        grid=grid,
        in_specs=[
            pl.BlockSpec((1, RB, FH), lambda i, c: (c, i, 0)),
            pl.BlockSpec((RB, 1), lambda i, c: (i, 0)),
            pl.BlockSpec((FH,), lambda i, c: (c,)),
        ],
        out_specs=pl.BlockSpec((RB, FH), lambda i, c: (i, c)),
        out_shape=jax.ShapeDtypeStruct((N_NODES, F), jnp.float32),
        name="gcn_out_scale_tc",
    )(acc, dis, b)


def kernel(x, edge_index, W, b):
    src = edge_index[0].astype(jnp.int32)
    dst = edge_index[1].astype(jnp.int32)
    degp = _deg_partial(dst)
    dis = _dis_from_partials(degp)
    hp = _matmul_scale(x, W, dis)
    acc = _edge_scatter(hp.reshape(NC * N_NODES, FH), src, dst)
    return _out_scale(acc, dis, b)


# speedup vs baseline: 1.1721x; 1.0418x over previous
import jax
import jax.numpy as jnp
from jax import lax
from jax.experimental import pallas as pl
from jax.experimental.pallas import tpu as pltpu, tpu_sc as plsc

N_NODES = 10000
N_PAD = 10240
E = 160000
F = 256
FH = 128
NC, NS = 2, 16
EPW = E // (NC * NS)
EPS = E // NS
CH = 80
NCHUNK = EPS // CH
RPT = N_PAD // NS
RCH = 16
VEC = 16
NB = 3
LOOK = 2


def _deg_body(dst_hbm, degp_hbm, dbuf, cnt, sem):
    c = lax.axis_index("c")
    s = lax.axis_index("s")
    wid = s * NC + c
    ones = jnp.ones((VEC,), jnp.float32)
    lane = lax.iota(jnp.int32, VEC)

    def zero(i, _):
        cnt[pl.ds(pl.multiple_of(i * VEC, VEC), VEC)] = jnp.zeros(
            (VEC,), jnp.float32)
        return 0
    lax.fori_loop(0, N_PAD // VEC, zero, 0)

    pltpu.sync_copy(dst_hbm.at[pl.ds(wid * EPW, EPW)], dbuf)

    nfull = EPW // VEC
    def body(i, _):
        idx = dbuf[pl.ds(pl.multiple_of(i * VEC, VEC), VEC)]
        plsc.addupdate_scatter(cnt, [idx], ones)
        return 0
    lax.fori_loop(0, nfull, body, 0)

    tail = EPW - nfull * VEC
    idx = dbuf[pl.ds(nfull * VEC - (VEC - tail), VEC)]
    mask = lane >= (VEC - tail)
    plsc.addupdate_scatter(cnt, [idx], ones, mask=mask)

    pltpu.sync_copy(cnt, degp_hbm.at[wid])


def _deg_partial(dst):
    mesh = plsc.VectorSubcoreMesh(core_axis_name="c", subcore_axis_name="s")
    return pl.kernel(
        _deg_body,
        out_type=jax.ShapeDtypeStruct((NC * NS, N_PAD), jnp.float32),
        mesh=mesh,
        scratch_types=[
            pltpu.VMEM((EPW,), jnp.int32),
            pltpu.VMEM((N_PAD,), jnp.float32),
            pltpu.SemaphoreType.DMA,
        ],
        compiler_params=pltpu.CompilerParams(needs_layout_passes=False),
        name="gcn_degree_sc",
    )(dst)


RBB = 512
RB = 400

def _mm_body(x_ref, w_ref, degp_ref, hp_ref, dis_ref):
    deg = jnp.sum(degp_ref[...], axis=0)
    dis = jnp.where(deg > 0.0, lax.rsqrt(jnp.maximum(deg, 1e-12)), 0.0)
    h = jnp.dot(x_ref[...], w_ref[...], preferred_element_type=jnp.float32)
    hp_ref[0] = h * dis[:, None]
    dis_ref[...] = dis[:, None]


def _matmul_scale(xp, W, degp):
    grid = (N_PAD // RBB, NC)
    return pl.pallas_call(
        _mm_body,
        grid=grid,
        in_specs=[
            pl.BlockSpec((RBB, F), lambda i, c: (i, 0)),
            pl.BlockSpec((F, FH), lambda i, c: (0, c)),
            pl.BlockSpec((NC * NS, RBB), lambda i, c: (0, i)),
        ],
        out_specs=[
            pl.BlockSpec((1, RBB, FH), lambda i, c: (c, i, 0)),
            pl.BlockSpec((RBB, 1), lambda i, c: (i, 0)),
        ],
        out_shape=[
            jax.ShapeDtypeStruct((NC, N_PAD, FH), jnp.float32),
            jax.ShapeDtypeStruct((N_PAD, 1), jnp.float32),
        ],
        name="gcn_matmul_scale_tc",
    )(xp, W, degp)


def _scatter_body(hp_hbm, src_hbm, dst_hbm, acc_hbm,
                  src_all, idx_b, dst_b, rows_b, stage, acc_sh,
                  gsem, dsem, ssem):
    c = lax.axis_index("c")
    s = lax.axis_index("s")
    base = s * EPS
    off = (c * N_PAD).astype(jnp.int32)

    pltpu.sync_copy(src_hbm.at[pl.ds(base, EPS)], src_all)

    def zstage(i, _):
        for v in range(FH // VEC):
            stage[i, pl.ds(v * VEC, VEC)] = jnp.zeros((VEC,), jnp.float32)
        return 0
    lax.fori_loop(0, RCH, zstage, 0)
    row0 = s * RPT
    for k in range(RPT // RCH):
        pltpu.sync_copy(stage, acc_sh.at[pl.ds(row0 + k * RCH, RCH)])
    plsc.subcore_barrier()

    def prep_issue(jj, b):
        e0 = pl.multiple_of(jj * CH, VEC)
        for v in range(CH // VEC):
            idx_b[b][pl.ds(v * VEC, VEC)] = (
                src_all[pl.ds(e0 + v * VEC, VEC)] + off)
        pltpu.async_copy(dst_hbm.at[pl.ds(base + e0, CH)], dst_b[b], dsem[b])
        pltpu.async_copy(hp_hbm.at[idx_b[b]], rows_b[b], gsem[b])

    for j in range(LOOK):
        prep_issue(jnp.int32(j), j)

    def consume(jj, b):
        pltpu.make_async_copy(hp_hbm.at[idx_b[b]], rows_b[b], gsem[b]).wait()
        pltpu.make_async_copy(
            dst_hbm.at[pl.ds(base, CH)], dst_b[b], dsem[b]).wait()
        pltpu.async_copy(rows_b[b], acc_sh.at[dst_b[b]], ssem[b], add=True)

    def step(t, _):
        for b0 in range(NB):
            jj = t * NB + b0

            @pl.when(jj < NCHUNK)
            def _():
                consume(jj, b0)
                nxt = jj + LOOK

                @pl.when(nxt < NCHUNK)
                def _():
                    b2 = (b0 + LOOK) % NB

                    @pl.when(nxt >= NB)
                    def _():
                        pltpu.make_async_copy(
                            rows_b[b2], acc_sh.at[dst_b[b2]],
                            ssem[b2]).wait()
                    prep_issue(nxt, b2)
        return 0
    lax.fori_loop(0, (NCHUNK + NB - 1) // NB, step, 0)

    for b in range(NB):
        pltpu.make_async_copy(rows_b[b], acc_sh.at[dst_b[b]], ssem[b]).wait()

    plsc.subcore_barrier()
    for k in range(RPT // RCH):
        r = row0 + k * RCH
        pltpu.sync_copy(acc_sh.at[pl.ds(r, RCH)], stage)
        pltpu.sync_copy(stage, acc_hbm.at[c, pl.ds(r, RCH)])


def _edge_scatter(hp_flat, src, dst):
    mesh = plsc.VectorSubcoreMesh(core_axis_name="c", subcore_axis_name="s")
    return pl.kernel(
        _scatter_body,
        out_type=jax.ShapeDtypeStruct((NC, N_PAD, FH), jnp.float32),
        mesh=mesh,
        scratch_types=[
            pltpu.VMEM((EPS,), jnp.int32),
            [pltpu.VMEM((CH,), jnp.int32) for _ in range(NB)],
            [pltpu.VMEM((CH,), jnp.int32) for _ in range(NB)],
            [pltpu.VMEM((CH, FH), jnp.float32) for _ in range(NB)],
            pltpu.VMEM((RCH, FH), jnp.float32),
            pltpu.VMEM_SHARED((N_PAD, FH), jnp.float32),
            [pltpu.SemaphoreType.DMA for _ in range(NB)],
            [pltpu.SemaphoreType.DMA for _ in range(NB)],
            [pltpu.SemaphoreType.DMA for _ in range(NB)],
        ],
        compiler_params=pltpu.CompilerParams(needs_layout_passes=False),
        name="gcn_edge_scatter_sc",
    )(hp_flat, src, dst)


def _out_body(acc_ref, dis_ref, b_ref, out_ref):
    out_ref[...] = acc_ref[0] * dis_ref[...] + b_ref[...][None, :]


def _out_scale(acc, dis, b):
    grid = (N_NODES // RB, NC)
    return pl.pallas_call(
        _out_body,
        grid=grid,
        in_specs=[
            pl.BlockSpec((1, RB, FH), lambda i, c: (c, i, 0)),
            pl.BlockSpec((RB, 1), lambda i, c: (i, 0)),
            pl.BlockSpec((FH,), lambda i, c: (c,)),
        ],
        out_specs=pl.BlockSpec((RB, FH), lambda i, c: (i, c)),
        out_shape=jax.ShapeDtypeStruct((N_NODES, F), jnp.float32),
        name="gcn_out_scale_tc",
    )(acc, dis, b)


def kernel(x, edge_index, W, b):
    src = edge_index[0].astype(jnp.int32)
    dst = edge_index[1].astype(jnp.int32)
    xp = jnp.pad(x, ((0, N_PAD - N_NODES), (0, 0)))
    degp = _deg_partial(dst)
    hp, dis = _matmul_scale(xp, W, degp)
    acc = _edge_scatter(hp.reshape(NC * N_PAD, FH), src, dst)
    return _out_scale(acc, dis, b)


# speedup vs baseline: 1.1791x; 1.0060x over previous
import jax
import jax.numpy as jnp
from jax import lax
from jax.experimental import pallas as pl
from jax.experimental.pallas import tpu as pltpu, tpu_sc as plsc

N_NODES = 10000
N_PAD = 10240
E = 160000
F = 256
FH = 128
NC, NS = 2, 16
EPW = E // (NC * NS)
EPS = E // NS
CH = 80
NCHUNK = EPS // CH
RPT = N_PAD // NS
RCH = 16
VEC = 16
NB = 3
LOOK = 2


def _deg_body(dst_hbm, degp_hbm, dbuf, cnt, sem):
    c = lax.axis_index("c")
    s = lax.axis_index("s")
    wid = s * NC + c
    ones = jnp.ones((VEC,), jnp.float32)
    lane = lax.iota(jnp.int32, VEC)

    def zero(i, _):
        cnt[pl.ds(pl.multiple_of(i * VEC, VEC), VEC)] = jnp.zeros(
            (VEC,), jnp.float32)
        return 0
    lax.fori_loop(0, N_PAD // VEC, zero, 0)

    pltpu.sync_copy(dst_hbm.at[pl.ds(wid * EPW, EPW)], dbuf)

    nfull = EPW // VEC
    def body(i, _):
        idx = dbuf[pl.ds(pl.multiple_of(i * VEC, VEC), VEC)]
        plsc.addupdate_scatter(cnt, [idx], ones)
        return 0
    lax.fori_loop(0, nfull, body, 0)

    tail = EPW - nfull * VEC
    idx = dbuf[pl.ds(nfull * VEC - (VEC - tail), VEC)]
    mask = lane >= (VEC - tail)
    plsc.addupdate_scatter(cnt, [idx], ones, mask=mask)

    pltpu.sync_copy(cnt, degp_hbm.at[wid])


def _deg_partial(dst):
    mesh = plsc.VectorSubcoreMesh(core_axis_name="c", subcore_axis_name="s")
    return pl.kernel(
        _deg_body,
        out_type=jax.ShapeDtypeStruct((NC * NS, N_PAD), jnp.float32),
        mesh=mesh,
        scratch_types=[
            pltpu.VMEM((EPW,), jnp.int32),
            pltpu.VMEM((N_PAD,), jnp.float32),
            pltpu.SemaphoreType.DMA,
        ],
        compiler_params=pltpu.CompilerParams(needs_layout_passes=False),
        name="gcn_degree_sc",
    )(dst)


RBB = 512
RB = 400

def _mm_body(x_ref, w_ref, degp_ref, hp_ref, dis_ref):
    deg = jnp.sum(degp_ref[...], axis=0)
    dis = jnp.where(deg > 0.0, lax.rsqrt(jnp.maximum(deg, 1e-12)), 0.0)
    h = jnp.dot(x_ref[...], w_ref[...], preferred_element_type=jnp.float32)
    hp_ref[0] = jnp.where(dis[:, None] > 0.0, h * dis[:, None], 0.0)
    dis_ref[...] = dis[:, None]


def _matmul_scale(xp, W, degp):
    grid = (N_PAD // RBB, NC)
    return pl.pallas_call(
        _mm_body,
        grid=grid,
        in_specs=[
            pl.BlockSpec((RBB, F), lambda i, c: (i, 0)),
            pl.BlockSpec((F, FH), lambda i, c: (0, c)),
            pl.BlockSpec((NC * NS, RBB), lambda i, c: (0, i)),
        ],
        out_specs=[
            pl.BlockSpec((1, RBB, FH), lambda i, c: (c, i, 0)),
            pl.BlockSpec((RBB, 1), lambda i, c: (i, 0)),
        ],
        out_shape=[
            jax.ShapeDtypeStruct((NC, N_PAD, FH), jnp.float32),
            jax.ShapeDtypeStruct((N_PAD, 1), jnp.float32),
        ],
        name="gcn_matmul_scale_tc",
    )(xp, W, degp)


def _scatter_body(hp_hbm, src_hbm, dst_hbm, acc_hbm,
                  src_all, idx_b, dst_b, rows_b, stage, acc_sh,
                  gsem, dsem, ssem):
    c = lax.axis_index("c")
    s = lax.axis_index("s")
    base = s * EPS
    off = (c * N_PAD).astype(jnp.int32)

    pltpu.sync_copy(src_hbm.at[pl.ds(base, EPS)], src_all)

    def zstage(i, _):
        for v in range(FH // VEC):
            stage[i, pl.ds(v * VEC, VEC)] = jnp.zeros((VEC,), jnp.float32)
        return 0
    lax.fori_loop(0, RCH, zstage, 0)
    row0 = s * RPT
    for k in range(RPT // RCH):
        pltpu.sync_copy(stage, acc_sh.at[pl.ds(row0 + k * RCH, RCH)])
    plsc.subcore_barrier()

    def prep_issue(jj, b):
        e0 = pl.multiple_of(jj * CH, VEC)
        for v in range(CH // VEC):
            idx_b[b][pl.ds(v * VEC, VEC)] = (
                src_all[pl.ds(e0 + v * VEC, VEC)] + off)
        pltpu.async_copy(dst_hbm.at[pl.ds(base + e0, CH)], dst_b[b], dsem[b])
        pltpu.async_copy(hp_hbm.at[idx_b[b]], rows_b[b], gsem[b])

    for j in range(LOOK):
        prep_issue(jnp.int32(j), j)

    def consume(jj, b):
        pltpu.make_async_copy(hp_hbm.at[idx_b[b]], rows_b[b], gsem[b]).wait()
        pltpu.make_async_copy(
            dst_hbm.at[pl.ds(base, CH)], dst_b[b], dsem[b]).wait()
        pltpu.async_copy(rows_b[b], acc_sh.at[dst_b[b]], ssem[b], add=True)

    def step(t, _):
        for b0 in range(NB):
            jj = t * NB + b0

            @pl.when(jj < NCHUNK)
            def _():
                consume(jj, b0)
                nxt = jj + LOOK

                @pl.when(nxt < NCHUNK)
                def _():
                    b2 = (b0 + LOOK) % NB

                    @pl.when(nxt >= NB)
                    def _():
                        pltpu.make_async_copy(
                            rows_b[b2], acc_sh.at[dst_b[b2]],
                            ssem[b2]).wait()
                    prep_issue(nxt, b2)
        return 0
    lax.fori_loop(0, (NCHUNK + NB - 1) // NB, step, 0)

    for b in range(NB):
        pltpu.make_async_copy(rows_b[b], acc_sh.at[dst_b[b]], ssem[b]).wait()

    plsc.subcore_barrier()
    for k in range(RPT // RCH):
        r = row0 + k * RCH
        pltpu.sync_copy(acc_sh.at[pl.ds(r, RCH)], stage)
        pltpu.sync_copy(stage, acc_hbm.at[c, pl.ds(r, RCH)])


def _edge_scatter(hp_flat, src, dst):
    mesh = plsc.VectorSubcoreMesh(core_axis_name="c", subcore_axis_name="s")
    return pl.kernel(
        _scatter_body,
        out_type=jax.ShapeDtypeStruct((NC, N_PAD, FH), jnp.float32),
        mesh=mesh,
        scratch_types=[
            pltpu.VMEM((EPS,), jnp.int32),
            [pltpu.VMEM((CH,), jnp.int32) for _ in range(NB)],
            [pltpu.VMEM((CH,), jnp.int32) for _ in range(NB)],
            [pltpu.VMEM((CH, FH), jnp.float32) for _ in range(NB)],
            pltpu.VMEM((RCH, FH), jnp.float32),
            pltpu.VMEM_SHARED((N_PAD, FH), jnp.float32),
            [pltpu.SemaphoreType.DMA for _ in range(NB)],
            [pltpu.SemaphoreType.DMA for _ in range(NB)],
            [pltpu.SemaphoreType.DMA for _ in range(NB)],
        ],
        compiler_params=pltpu.CompilerParams(needs_layout_passes=False),
        name="gcn_edge_scatter_sc",
    )(hp_flat, src, dst)


def _out_body(acc_ref, dis_ref, b_ref, out_ref):
    out_ref[...] = acc_ref[0] * dis_ref[...] + b_ref[...][None, :]


def _out_scale(acc, dis, b):
    grid = (N_NODES // RB, NC)
    return pl.pallas_call(
        _out_body,
        grid=grid,
        in_specs=[
            pl.BlockSpec((1, RB, FH), lambda i, c: (c, i, 0)),
            pl.BlockSpec((RB, 1), lambda i, c: (i, 0)),
            pl.BlockSpec((FH,), lambda i, c: (c,)),
        ],
        out_specs=pl.BlockSpec((RB, FH), lambda i, c: (i, c)),
        out_shape=jax.ShapeDtypeStruct((N_NODES, F), jnp.float32),
        name="gcn_out_scale_tc",
    )(acc, dis, b)


def kernel(x, edge_index, W, b):
    src = edge_index[0].astype(jnp.int32)
    dst = edge_index[1].astype(jnp.int32)
    degp = _deg_partial(dst)
    hp, dis = _matmul_scale(x, W, degp)
    acc = _edge_scatter(hp.reshape(NC * N_PAD, FH), src, dst)
    return _out_scale(acc, dis, b)


# speedup vs baseline: 1.1905x; 1.0096x over previous
import jax
import jax.numpy as jnp
from jax import lax
from jax.experimental import pallas as pl
from jax.experimental.pallas import tpu as pltpu, tpu_sc as plsc

N_NODES = 10000
N_PAD = 10240
E = 160000
F = 256
FH = 128
NC, NS = 2, 16
EPW = E // (NC * NS)
EPS = E // NS
CH = 80
NCHUNK = EPS // CH
RPT = N_PAD // NS
RCH = 16
VEC = 16
NB = 3
LOOK = 2


def _deg_body(dst_hbm, degp_hbm, dbuf, cnt, sem):
    c = lax.axis_index("c")
    s = lax.axis_index("s")
    wid = s * NC + c
    ones = jnp.ones((VEC,), jnp.float32)
    lane = lax.iota(jnp.int32, VEC)

    def zero(i, _):
        cnt[pl.ds(pl.multiple_of(i * VEC, VEC), VEC)] = jnp.zeros(
            (VEC,), jnp.float32)
        return 0
    lax.fori_loop(0, N_PAD // VEC, zero, 0)

    pltpu.sync_copy(dst_hbm.at[pl.ds(wid * EPW, EPW)], dbuf)

    nfull = EPW // VEC
    def body(i, _):
        idx = dbuf[pl.ds(pl.multiple_of(i * VEC, VEC), VEC)]
        plsc.addupdate_scatter(cnt, [idx], ones)
        return 0
    lax.fori_loop(0, nfull, body, 0)

    tail = EPW - nfull * VEC
    idx = dbuf[pl.ds(nfull * VEC - (VEC - tail), VEC)]
    mask = lane >= (VEC - tail)
    plsc.addupdate_scatter(cnt, [idx], ones, mask=mask)

    pltpu.sync_copy(cnt, degp_hbm.at[wid])


def _deg_partial(dst):
    mesh = plsc.VectorSubcoreMesh(core_axis_name="c", subcore_axis_name="s")
    return pl.kernel(
        _deg_body,
        out_type=jax.ShapeDtypeStruct((NC * NS, N_PAD), jnp.float32),
        mesh=mesh,
        scratch_types=[
            pltpu.VMEM((EPW,), jnp.int32),
            pltpu.VMEM((N_PAD,), jnp.float32),
            pltpu.SemaphoreType.DMA,
        ],
        compiler_params=pltpu.CompilerParams(needs_layout_passes=False),
        name="gcn_degree_sc",
    )(dst)


RBB = 512
RB = 400

def _mm_body(x_ref, w_ref, degp_ref, hp_ref, dis_ref):
    deg = jnp.sum(degp_ref[...], axis=0)
    dis = jnp.where(deg > 0.0, lax.rsqrt(jnp.maximum(deg, 1e-12)), 0.0)
    h = jnp.dot(x_ref[...], w_ref[...], preferred_element_type=jnp.float32)
    hp_ref[0] = jnp.where(dis[:, None] > 0.0, h * dis[:, None], 0.0)
    dis_ref[...] = dis[:, None]


def _matmul_scale(xp, W, degp):
    grid = (N_PAD // RBB, NC)
    return pl.pallas_call(
        _mm_body,
        grid=grid,
        in_specs=[
            pl.BlockSpec((RBB, F), lambda i, c: (i, 0)),
            pl.BlockSpec((F, FH), lambda i, c: (0, c)),
            pl.BlockSpec((NC * NS, RBB), lambda i, c: (0, i)),
        ],
        out_specs=[
            pl.BlockSpec((1, RBB, FH), lambda i, c: (c, i, 0)),
            pl.BlockSpec((RBB, 1), lambda i, c: (i, 0)),
        ],
        out_shape=[
            jax.ShapeDtypeStruct((NC, N_PAD, FH), jnp.float32),
            jax.ShapeDtypeStruct((N_PAD, 1), jnp.float32),
        ],
        name="gcn_matmul_scale_tc",
    )(xp, W, degp)


def _scatter_body(hp_hbm, srcoff_hbm, dst_hbm, acc_hbm,
                  src_all, dst_b, rows_b, stage, acc_sh,
                  gsem, dsem, ssem):
    c = lax.axis_index("c")
    s = lax.axis_index("s")
    base = s * EPS

    pltpu.sync_copy(srcoff_hbm.at[pl.ds(c * E + base, EPS)], src_all)

    def zstage(i, _):
        for v in range(FH // VEC):
            stage[i, pl.ds(v * VEC, VEC)] = jnp.zeros((VEC,), jnp.float32)
        return 0
    lax.fori_loop(0, RCH, zstage, 0)
    row0 = s * RPT
    for k in range(RPT // RCH):
        pltpu.sync_copy(stage, acc_sh.at[pl.ds(row0 + k * RCH, RCH)])
    plsc.subcore_barrier()

    def prep_issue(jj, b):
        e0 = pl.multiple_of(jj * CH, VEC)
        pltpu.async_copy(dst_hbm.at[pl.ds(base + e0, CH)], dst_b[b], dsem[b])
        pltpu.async_copy(hp_hbm.at[src_all.at[pl.ds(e0, CH)]],
                         rows_b[b], gsem[b])

    for j in range(LOOK):
        prep_issue(jnp.int32(j), j)

    def consume(jj, b):
        e0 = pl.multiple_of(jj * CH, VEC)
        pltpu.make_async_copy(hp_hbm.at[src_all.at[pl.ds(e0, CH)]],
                              rows_b[b], gsem[b]).wait()
        pltpu.make_async_copy(
            dst_hbm.at[pl.ds(base, CH)], dst_b[b], dsem[b]).wait()
        pltpu.async_copy(rows_b[b], acc_sh.at[dst_b[b]], ssem[b], add=True)

    def step(t, _):
        for b0 in range(NB):
            jj = t * NB + b0

            @pl.when(jj < NCHUNK)
            def _():
                consume(jj, b0)
                nxt = jj + LOOK

                @pl.when(nxt < NCHUNK)
                def _():
                    b2 = (b0 + LOOK) % NB

                    @pl.when(nxt >= NB)
                    def _():
                        pltpu.make_async_copy(
                            rows_b[b2], acc_sh.at[dst_b[b2]],
                            ssem[b2]).wait()
                    prep_issue(nxt, b2)
        return 0
    lax.fori_loop(0, (NCHUNK + NB - 1) // NB, step, 0)

    for b in range(NB):
        pltpu.make_async_copy(rows_b[b], acc_sh.at[dst_b[b]], ssem[b]).wait()

    plsc.subcore_barrier()
    for k in range(RPT // RCH):
        r = row0 + k * RCH
        pltpu.sync_copy(acc_sh.at[pl.ds(r, RCH)], stage)
        pltpu.sync_copy(stage, acc_hbm.at[c, pl.ds(r, RCH)])


def _edge_scatter(hp_flat, srcoff, dst):
    mesh = plsc.VectorSubcoreMesh(core_axis_name="c", subcore_axis_name="s")
    return pl.kernel(
        _scatter_body,
        out_type=jax.ShapeDtypeStruct((NC, N_PAD, FH), jnp.float32),
        mesh=mesh,
        scratch_types=[
            pltpu.VMEM((EPS,), jnp.int32),
            [pltpu.VMEM((CH,), jnp.int32) for _ in range(NB)],
            [pltpu.VMEM((CH, FH), jnp.float32) for _ in range(NB)],
            pltpu.VMEM((RCH, FH), jnp.float32),
            pltpu.VMEM_SHARED((N_PAD, FH), jnp.float32),
            [pltpu.SemaphoreType.DMA for _ in range(NB)],
            [pltpu.SemaphoreType.DMA for _ in range(NB)],
            [pltpu.SemaphoreType.DMA for _ in range(NB)],
        ],
        compiler_params=pltpu.CompilerParams(needs_layout_passes=False),
        name="gcn_edge_scatter_sc",
    )(hp_flat, srcoff, dst)


def _out_body(acc_ref, dis_ref, b_ref, out_ref):
    out_ref[...] = acc_ref[0] * dis_ref[...] + b_ref[...][None, :]


def _out_scale(acc, dis, b):
    grid = (N_NODES // RB, NC)
    return pl.pallas_call(
        _out_body,
        grid=grid,
        in_specs=[
            pl.BlockSpec((1, RB, FH), lambda i, c: (c, i, 0)),
            pl.BlockSpec((RB, 1), lambda i, c: (i, 0)),
            pl.BlockSpec((FH,), lambda i, c: (c,)),
        ],
        out_specs=pl.BlockSpec((RB, FH), lambda i, c: (i, c)),
        out_shape=jax.ShapeDtypeStruct((N_NODES, F), jnp.float32),
        name="gcn_out_scale_tc",
    )(acc, dis, b)


def kernel(x, edge_index, W, b):
    src = edge_index[0].astype(jnp.int32)
    dst = edge_index[1].astype(jnp.int32)
    degp = _deg_partial(dst)
    hp, dis = _matmul_scale(x, W, degp)
    srcoff = jnp.concatenate([src, src + N_PAD])
    acc = _edge_scatter(hp.reshape(NC * N_PAD, FH), srcoff, dst)
    return _out_scale(acc, dis, b)


# speedup vs baseline: 1.2171x; 1.0224x over previous
import jax
import jax.numpy as jnp
from jax import lax
from jax.experimental import pallas as pl
from jax.experimental.pallas import tpu as pltpu, tpu_sc as plsc

N_NODES = 10000
N_PAD = 10240
E = 160000
F = 256
FH = 128
NC, NS = 2, 16
EPW = E // (NC * NS)
EPS = E // NS
CH = 80
NCHUNK = EPS // CH
RPT = N_PAD // NS
RCH = 32
VEC = 16
NB = 3
LOOK = 2


def _deg_body(dst_hbm, degp_hbm, dbuf, cnt, sem):
    c = lax.axis_index("c")
    s = lax.axis_index("s")
    wid = s * NC + c
    ones = jnp.ones((VEC,), jnp.float32)
    lane = lax.iota(jnp.int32, VEC)

    def zero(i, _):
        cnt[pl.ds(pl.multiple_of(i * VEC, VEC), VEC)] = jnp.zeros(
            (VEC,), jnp.float32)
        return 0
    lax.fori_loop(0, N_PAD // VEC, zero, 0)

    pltpu.sync_copy(dst_hbm.at[pl.ds(wid * EPW, EPW)], dbuf)

    nfull = EPW // VEC
    def body(i, _):
        idx = dbuf[pl.ds(pl.multiple_of(i * VEC, VEC), VEC)]
        plsc.addupdate_scatter(cnt, [idx], ones)
        return 0
    lax.fori_loop(0, nfull, body, 0)

    tail = EPW - nfull * VEC
    idx = dbuf[pl.ds(nfull * VEC - (VEC - tail), VEC)]
    mask = lane >= (VEC - tail)
    plsc.addupdate_scatter(cnt, [idx], ones, mask=mask)

    pltpu.sync_copy(cnt, degp_hbm.at[wid])


def _deg_partial(dst):
    mesh = plsc.VectorSubcoreMesh(core_axis_name="c", subcore_axis_name="s")
    return pl.kernel(
        _deg_body,
        out_type=jax.ShapeDtypeStruct((NC * NS, N_PAD), jnp.float32),
        mesh=mesh,
        scratch_types=[
            pltpu.VMEM((EPW,), jnp.int32),
            pltpu.VMEM((N_PAD,), jnp.float32),
            pltpu.SemaphoreType.DMA,
        ],
        compiler_params=pltpu.CompilerParams(needs_layout_passes=False),
        name="gcn_degree_sc",
    )(dst)


RBB = 512
RB = 400

def _mm_body(x_ref, w_ref, degp_ref, hp_ref, dis_ref):
    deg = jnp.sum(degp_ref[...], axis=0)
    dis = jnp.where(deg > 0.0, lax.rsqrt(jnp.maximum(deg, 1e-12)), 0.0)
    h = jnp.dot(x_ref[...], w_ref[...], preferred_element_type=jnp.float32)
    hp_ref[0] = jnp.where(dis[:, None] > 0.0, h * dis[:, None], 0.0)
    dis_ref[...] = dis[:, None]


def _matmul_scale(xp, W, degp):
    grid = (N_PAD // RBB, NC)
    return pl.pallas_call(
        _mm_body,
        grid=grid,
        in_specs=[
            pl.BlockSpec((RBB, F), lambda i, c: (i, 0)),
            pl.BlockSpec((F, FH), lambda i, c: (0, c)),
            pl.BlockSpec((NC * NS, RBB), lambda i, c: (0, i)),
        ],
        out_specs=[
            pl.BlockSpec((1, RBB, FH), lambda i, c: (c, i, 0)),
            pl.BlockSpec((RBB, 1), lambda i, c: (i, 0)),
        ],
        out_shape=[
            jax.ShapeDtypeStruct((NC, N_PAD, FH), jnp.float32),
            jax.ShapeDtypeStruct((N_PAD, 1), jnp.float32),
        ],
        name="gcn_matmul_scale_tc",
    )(xp, W, degp)


def _scatter_body(hp_hbm, srcoff_hbm, dst_hbm, acc_hbm,
                  src_all, dst_b, rows_b, stage, acc_sh,
                  gsem, dsem, ssem):
    c = lax.axis_index("c")
    s = lax.axis_index("s")
    base = s * EPS

    pltpu.sync_copy(srcoff_hbm.at[pl.ds(c * E + base, EPS)], src_all)

    def zstage(i, _):
        for v in range(FH // VEC):
            stage[i, pl.ds(v * VEC, VEC)] = jnp.zeros((VEC,), jnp.float32)
        return 0
    lax.fori_loop(0, RCH, zstage, 0)
    row0 = s * RPT
    for k in range(RPT // RCH):
        pltpu.sync_copy(stage, acc_sh.at[pl.ds(row0 + k * RCH, RCH)])
    plsc.subcore_barrier()

    def prep_issue(jj, b):
        e0 = pl.multiple_of(jj * CH, VEC)
        pltpu.async_copy(dst_hbm.at[pl.ds(base + e0, CH)], dst_b[b], dsem[b])
        pltpu.async_copy(hp_hbm.at[src_all.at[pl.ds(e0, CH)]],
                         rows_b[b], gsem[b])

    for j in range(LOOK):
        prep_issue(jnp.int32(j), j)

    def consume(jj, b):
        e0 = pl.multiple_of(jj * CH, VEC)
        pltpu.make_async_copy(hp_hbm.at[src_all.at[pl.ds(e0, CH)]],
                              rows_b[b], gsem[b]).wait()
        pltpu.make_async_copy(
            dst_hbm.at[pl.ds(base, CH)], dst_b[b], dsem[b]).wait()
        pltpu.async_copy(rows_b[b], acc_sh.at[dst_b[b]], ssem[b], add=True)

    def step(t, _):
        for b0 in range(NB):
            jj = t * NB + b0

            @pl.when(jj < NCHUNK)
            def _():
                consume(jj, b0)
                nxt = jj + LOOK

                @pl.when(nxt < NCHUNK)
                def _():
                    b2 = (b0 + LOOK) % NB

                    @pl.when(nxt >= NB)
                    def _():
                        pltpu.make_async_copy(
                            rows_b[b2], acc_sh.at[dst_b[b2]],
                            ssem[b2]).wait()
                    prep_issue(nxt, b2)
        return 0
    lax.fori_loop(0, (NCHUNK + NB - 1) // NB, step, 0)

    for b in range(NB):
        pltpu.make_async_copy(rows_b[b], acc_sh.at[dst_b[b]], ssem[b]).wait()

    plsc.subcore_barrier()
    for k in range(RPT // RCH):
        r = row0 + k * RCH
        pltpu.sync_copy(acc_sh.at[pl.ds(r, RCH)], stage)
        pltpu.sync_copy(stage, acc_hbm.at[c, pl.ds(r, RCH)])


def _edge_scatter(hp_flat, srcoff, dst):
    mesh = plsc.VectorSubcoreMesh(core_axis_name="c", subcore_axis_name="s")
    return pl.kernel(
        _scatter_body,
        out_type=jax.ShapeDtypeStruct((NC, N_PAD, FH), jnp.float32),
        mesh=mesh,
        scratch_types=[
            pltpu.VMEM((EPS,), jnp.int32),
            [pltpu.VMEM((CH,), jnp.int32) for _ in range(NB)],
            [pltpu.VMEM((CH, FH), jnp.float32) for _ in range(NB)],
            pltpu.VMEM((RCH, FH), jnp.float32),
            pltpu.VMEM_SHARED((N_PAD, FH), jnp.float32),
            [pltpu.SemaphoreType.DMA for _ in range(NB)],
            [pltpu.SemaphoreType.DMA for _ in range(NB)],
            [pltpu.SemaphoreType.DMA for _ in range(NB)],
        ],
        compiler_params=pltpu.CompilerParams(needs_layout_passes=False),
        name="gcn_edge_scatter_sc",
    )(hp_flat, srcoff, dst)


def _out_body(acc_ref, dis_ref, b_ref, out_ref):
    out_ref[...] = acc_ref[0] * dis_ref[...] + b_ref[...][None, :]


def _out_scale(acc, dis, b):
    grid = (N_NODES // RB, NC)
    return pl.pallas_call(
        _out_body,
        grid=grid,
        in_specs=[
            pl.BlockSpec((1, RB, FH), lambda i, c: (c, i, 0)),
            pl.BlockSpec((RB, 1), lambda i, c: (i, 0)),
            pl.BlockSpec((FH,), lambda i, c: (c,)),
        ],
        out_specs=pl.BlockSpec((RB, FH), lambda i, c: (i, c)),
        out_shape=jax.ShapeDtypeStruct((N_NODES, F), jnp.float32),
        name="gcn_out_scale_tc",
    )(acc, dis, b)


def kernel(x, edge_index, W, b):
    src = edge_index[0].astype(jnp.int32)
    dst = edge_index[1].astype(jnp.int32)
    degp = _deg_partial(dst)
    hp, dis = _matmul_scale(x, W, degp)
    srcoff = jnp.concatenate([src, src + N_PAD])
    acc = _edge_scatter(hp.reshape(NC * N_PAD, FH), srcoff, dst)
    return _out_scale(acc, dis, b)
